# Initial kernel scaffold; baseline (speedup 1.0000x reference)
import jax
import jax.numpy as jnp
from jax.experimental import pallas as pl


def kernel(x, edge_index, edge_attr, params, scalars):
    raise NotImplementedError("write your pallas kernel here")


# speedup vs baseline: 1.1026x; 1.1026x over previous
import functools

import jax
import jax.numpy as jnp
from jax import lax
from jax.experimental import pallas as pl
from jax.experimental.pallas import tpu as pltpu
from jax.experimental.pallas import tpu_sc as plsc

_N = 10000
_E = 160000
_NP = 10240
_C = 64
_NCHUNK = _NP // _C
_NW = 32
_CPW = _NCHUNK // _NW
_G = 32
_CPTR_PAD = 192


def _mm_bias(a, w, b):
    M, K = a.shape
    Do = w.shape[1]
    BM = 256
    if b is None:
        b = jnp.zeros((1, Do), jnp.float32)
    else:
        b = b.reshape(1, Do)

    def kern(a_ref, w_ref, b_ref, o_ref):
        o_ref[...] = jnp.dot(a_ref[...], w_ref[...],
                             preferred_element_type=jnp.float32) + b_ref[...]

    return pl.pallas_call(
        kern,
        grid=(M // BM,),
        in_specs=[
            pl.BlockSpec((BM, K), lambda i: (i, 0)),
            pl.BlockSpec((K, Do), lambda i: (0, 0)),
            pl.BlockSpec((1, Do), lambda i: (0, 0)),
        ],
        out_specs=pl.BlockSpec((BM, Do), lambda i: (i, 0)),
        out_shape=jax.ShapeDtypeStruct((M, Do), jnp.float32),
    )(a, w, b)


def _wsum(coefs, arrays):
    M, Do = arrays[0].shape
    BM = 512
    k = len(arrays)
    cof = jnp.zeros((8,), jnp.float32).at[:k].set(jnp.stack(coefs))

    def kern(c_ref, *refs):
        o_ref = refs[-1]
        acc = c_ref[0] * refs[0][...]
        for i in range(1, k):
            acc = acc + c_ref[i] * refs[i][...]
        o_ref[...] = acc

    return pl.pallas_call(
        kern,
        grid=(M // BM,),
        in_specs=[pl.BlockSpec(memory_space=pltpu.SMEM)] +
                 [pl.BlockSpec((BM, Do), lambda i: (i, 0)) for _ in range(k)],
        out_specs=pl.BlockSpec((BM, Do), lambda i: (i, 0)),
        out_shape=jax.ShapeDtypeStruct((M, Do), jnp.float32),
    )(cof, *arrays)


def _sget(vref, i):
    v = vref[pl.ds(i, 16)]
    return v[0]


def _sc_edge_layer(xm, ea, xs, ssrc, sord, sdst, cptr, Do):
    mesh = plsc.VectorSubcoreMesh(core_axis_name="c", subcore_axis_name="s")

    @functools.partial(
        pl.kernel,
        out_type=jax.ShapeDtypeStruct((_NP, Do), jnp.float32),
        mesh=mesh,
        scratch_types=[
            pltpu.VMEM((_CPTR_PAD,), jnp.int32),
            pltpu.VMEM((_G,), jnp.int32),
            pltpu.VMEM((_G,), jnp.int32),
            pltpu.VMEM((_G + 16,), jnp.int32),
            pltpu.VMEM((_G, Do), jnp.float32),
            pltpu.VMEM((_G, Do), jnp.float32),
            pltpu.VMEM((_C, Do), jnp.float32),
            pltpu.VMEM((_C, Do), jnp.float32),
            pltpu.SemaphoreType.DMA,
            pltpu.SemaphoreType.DMA,
        ],
    )
    def k(xm_h, ea_h, xs_h, ssrc_h, sord_h, sdst_h, cptr_h, h_out,
          cptr_v, idx_s, idx_e, dst_v, xg, eg, acc, xsb, sem1, sem2):
        wid = lax.axis_index("c") * 16 + lax.axis_index("s")
        pltpu.sync_copy(cptr_h, cptr_v)
        for t in range(_CPW):
            chunk = wid * _CPW + t
            base_row = chunk * _C
            e_lo = _sget(cptr_v, chunk)
            e_hi = _sget(cptr_v, chunk + 1)

            def zbody(r, _):
                for j in range(Do // 16):
                    acc[r, pl.ds(j * 16, 16)] = jnp.zeros((16,), jnp.float32)
                return 0
            lax.fori_loop(0, _C, zbody, 0)

            g_lo = e_lo // _G
            g_hi = (e_hi + _G - 1) // _G

            def gbody(g, _):
                off = g * _G
                pltpu.sync_copy(ssrc_h.at[pl.ds(off, _G)], idx_s)
                pltpu.sync_copy(sord_h.at[pl.ds(off, _G)], idx_e)
                pltpu.sync_copy(sdst_h.at[pl.ds(off, _G)], dst_v.at[pl.ds(0, _G)])
                c1 = pltpu.async_copy(xm_h.at[idx_s], xg, sem1)
                c2 = pltpu.async_copy(ea_h.at[idx_e], eg, sem2)
                c1.wait()
                c2.wait()
                lo = jnp.maximum(off, e_lo) - off
                hi = jnp.minimum(off + _G, e_hi) - off

                def ebody(t2, _):
                    row = _sget(dst_v, t2) - base_row
                    for j in range(Do // 16):
                        sl = pl.ds(j * 16, 16)
                        v = jnp.maximum(xg[t2, sl] + eg[t2, sl], 0.0)
                        plsc.addupdate(acc.at[row, sl], v)
                    return 0
                lax.fori_loop(lo, hi, ebody, 0)
                return 0
            lax.fori_loop(g_lo, g_hi, gbody, 0)

            pltpu.sync_copy(xs_h.at[pl.ds(base_row, _C)], xsb)

            def cbody(r, _):
                for j in range(Do // 16):
                    sl = pl.ds(j * 16, 16)
                    acc[r, sl] = jnp.maximum(acc[r, sl] + xsb[r, sl], 0.0)
                return 0
            lax.fori_loop(0, _C, cbody, 0)
            pltpu.sync_copy(acc, h_out.at[pl.ds(base_row, _C)])

    return k(xm, ea, xs, ssrc, sord, sdst, cptr)


def kernel(x, edge_index, edge_attr, params, scalars):
    src = edge_index[0].astype(jnp.int32)
    dst = edge_index[1].astype(jnp.int32)
    order = jnp.argsort(dst).astype(jnp.int32)
    sdst = jnp.take(dst, order)
    ssrc = jnp.take(src, order)
    bounds = jnp.arange(0, _NP + _C, _C, dtype=jnp.int32)
    cptr = jnp.searchsorted(sdst, bounds, side="left").astype(jnp.int32)
    cptr = jnp.pad(cptr, (0, _CPTR_PAD - cptr.shape[0]))
    xp = jnp.pad(x, ((0, _NP - _N), (0, 0)))
    s = scalars

    def layer(p, xin):
        Do = p["W_self"].shape[1]
        xm = _mm_bias(xin, p["W_msg"], p["b_msg"])
        xs = _mm_bias(xin, p["W_self"], p["b_self"])
        ea = _mm_bias(edge_attr, p["W_edge"], None)
        return _sc_edge_layer(xm, ea, xs, ssrc, order, sdst, cptr, Do)

    x1 = layer(params[0], xp)
    x2 = layer(params[1], x1)
    x2w = _wsum([s[0], s[1]], [x1, x2])
    x3 = layer(params[2], x2w)
    x3w = _wsum([s[2], s[3], s[4]], [x1, x2w, x3])
    x4 = layer(params[3], x3w)
    x4w = _wsum([s[5], s[6], s[7], s[8]], [x1, x2w, x3w, x4])
    x5 = layer(params[3], x4w)
    x5w = _wsum([s[9], s[10], s[11], s[12], s[13]], [x1, x2w, x3w, x4w, x5])
    out = layer(params[5], x5w)
    return out[:_N]


# speedup vs baseline: 1.4533x; 1.3181x over previous
import functools

import jax
import jax.numpy as jnp
from jax import lax
from jax.experimental import pallas as pl
from jax.experimental.pallas import tpu as pltpu
from jax.experimental.pallas import tpu_sc as plsc

_N = 10000
_E = 160000
_NP = 10240
_C = 64
_NCHUNK = _NP // _C
_NW = 32
_CPW = _NCHUNK // _NW
_G = 32
_CPTR_PAD = 192


def _mm_bias(a, w, b):
    M, K = a.shape
    Do = w.shape[1]
    BM = 256
    if b is None:
        b = jnp.zeros((1, Do), jnp.float32)
    else:
        b = b.reshape(1, Do)

    def kern(a_ref, w_ref, b_ref, o_ref):
        o_ref[...] = jnp.dot(a_ref[...], w_ref[...],
                             preferred_element_type=jnp.float32) + b_ref[...]

    return pl.pallas_call(
        kern,
        grid=(M // BM,),
        in_specs=[
            pl.BlockSpec((BM, K), lambda i: (i, 0)),
            pl.BlockSpec((K, Do), lambda i: (0, 0)),
            pl.BlockSpec((1, Do), lambda i: (0, 0)),
        ],
        out_specs=pl.BlockSpec((BM, Do), lambda i: (i, 0)),
        out_shape=jax.ShapeDtypeStruct((M, Do), jnp.float32),
    )(a, w, b)


def _wsum(coefs, arrays):
    M, Do = arrays[0].shape
    BM = 512
    k = len(arrays)
    cof = jnp.zeros((8,), jnp.float32).at[:k].set(jnp.stack(coefs))

    def kern(c_ref, *refs):
        o_ref = refs[-1]
        acc = c_ref[0] * refs[0][...]
        for i in range(1, k):
            acc = acc + c_ref[i] * refs[i][...]
        o_ref[...] = acc

    return pl.pallas_call(
        kern,
        grid=(M // BM,),
        in_specs=[pl.BlockSpec(memory_space=pltpu.SMEM)] +
                 [pl.BlockSpec((BM, Do), lambda i: (i, 0)) for _ in range(k)],
        out_specs=pl.BlockSpec((BM, Do), lambda i: (i, 0)),
        out_shape=jax.ShapeDtypeStruct((M, Do), jnp.float32),
    )(cof, *arrays)


def _sget(vref, i):
    v = vref[pl.ds(i, 16)]
    return v[0]


_NGS = 32
_SG = _NGS * _G
_SGP = _SG + 32


def _sc_edge_layer(xm, ea, xs, ssrc, sord, sdst, cptr, Do):
    mesh = plsc.VectorSubcoreMesh(core_axis_name="c", subcore_axis_name="s")

    @functools.partial(
        pl.kernel,
        out_type=jax.ShapeDtypeStruct((_NP, Do), jnp.float32),
        mesh=mesh,
        scratch_types=[
            pltpu.VMEM((_CPTR_PAD,), jnp.int32),
            pltpu.VMEM((_SGP,), jnp.int32),
            pltpu.VMEM((_SGP,), jnp.int32),
            pltpu.VMEM((_SGP,), jnp.int32),
            pltpu.VMEM((2, _G, Do), jnp.float32),
            pltpu.VMEM((2, _G, Do), jnp.float32),
            pltpu.VMEM((_C, Do), jnp.float32),
            pltpu.SemaphoreType.DMA,
            pltpu.SemaphoreType.DMA,
            pltpu.SemaphoreType.DMA,
            pltpu.SemaphoreType.DMA,
        ],
    )
    def k(xm_h, ea_h, xs_h, ssrc_h, sord_h, sdst_h, cptr_h, h_out,
          cptr_v, srcb, ordb, dstb, xg, eg, acc,
          semx0, seme0, semx1, seme1):
        wid = lax.axis_index("c") * 16 + lax.axis_index("s")
        pltpu.sync_copy(cptr_h, cptr_v)
        sems = ((semx0, seme0), (semx1, seme1))

        def issue(kk, slot):
            koff = kk * _G
            pltpu.async_copy(xm_h.at[srcb.at[pl.ds(koff, _G)]], xg.at[slot],
                             sems[slot][0])
            pltpu.async_copy(ea_h.at[ordb.at[pl.ds(koff, _G)]], eg.at[slot],
                             sems[slot][1])

        def wait(kk, slot):
            koff = kk * _G
            pltpu.make_async_copy(xm_h.at[srcb.at[pl.ds(koff, _G)]],
                                  xg.at[slot], sems[slot][0]).wait()
            pltpu.make_async_copy(ea_h.at[ordb.at[pl.ds(koff, _G)]],
                                  eg.at[slot], sems[slot][1]).wait()

        for t in range(_CPW):
            chunk = wid * _CPW + t
            base_row = chunk * _C
            e_lo = _sget(cptr_v, chunk)
            e_hi = _sget(cptr_v, chunk + 1)

            def zbody(r, _):
                for j in range(Do // 16):
                    acc[r, pl.ds(j * 16, 16)] = jnp.zeros((16,), jnp.float32)
                return 0
            lax.fori_loop(0, _C, zbody, 0)

            g_lo = e_lo // _G
            g_hi = (e_hi + _G - 1) // _G
            nsg = (g_hi - g_lo + _NGS - 1) // _NGS

            def sgbody(i, _):
                gstart = g_lo + i * _NGS
                gcnt = jnp.minimum(_NGS, g_hi - gstart)
                off = gstart * _G
                pltpu.sync_copy(ssrc_h.at[pl.ds(off, _SG)],
                                srcb.at[pl.ds(0, _SG)])
                pltpu.sync_copy(sord_h.at[pl.ds(off, _SG)],
                                ordb.at[pl.ds(0, _SG)])
                pltpu.sync_copy(sdst_h.at[pl.ds(off, _SG)],
                                dstb.at[pl.ds(0, _SG)])

                @pl.when(gcnt > 0)
                def _():
                    issue(0, 0)

                def compute(kk, slot):
                    koff = kk * _G
                    goff = off + koff
                    lo = jnp.maximum(goff, e_lo) - goff
                    hi = jnp.minimum(goff + _G, e_hi) - goff

                    def ebody(t2, _):
                        row = _sget(dstb, koff + t2) - base_row
                        for j in range(Do // 16):
                            sl = pl.ds(j * 16, 16)
                            v = jnp.maximum(xg[slot, t2, sl] + eg[slot, t2, sl],
                                            0.0)
                            plsc.addupdate(acc.at[row, sl], v)
                        return 0
                    lax.fori_loop(lo, hi, ebody, 0)

                def step(kk, slot):
                    wait(kk, slot)

                    @pl.when(kk + 1 < gcnt)
                    def _():
                        issue(kk + 1, 1 - slot)
                    compute(kk, slot)

                def kbody(kk, _):
                    @pl.when(kk % 2 == 0)
                    def _():
                        step(kk, 0)

                    @pl.when(kk % 2 == 1)
                    def _():
                        step(kk, 1)
                    return 0
                lax.fori_loop(0, gcnt, kbody, 0)
                return 0
            lax.fori_loop(0, nsg, sgbody, 0)

            pltpu.sync_copy(xs_h.at[pl.ds(base_row, _C // 2)], xg.at[0])
            pltpu.sync_copy(xs_h.at[pl.ds(base_row + _C // 2, _C // 2)],
                            xg.at[1])

            def cbody(r, _):
                b = r // (_C // 2)
                rr = r - b * (_C // 2)
                for j in range(Do // 16):
                    sl = pl.ds(j * 16, 16)
                    acc[r, sl] = jnp.maximum(acc[r, sl] + xg[b, rr, sl], 0.0)
                return 0
            lax.fori_loop(0, _C, cbody, 0)
            pltpu.sync_copy(acc, h_out.at[pl.ds(base_row, _C)])

    return k(xm, ea, xs, ssrc, sord, sdst, cptr)


def kernel(x, edge_index, edge_attr, params, scalars):
    src = edge_index[0].astype(jnp.int32)
    dst = edge_index[1].astype(jnp.int32)
    order = jnp.argsort(dst).astype(jnp.int32)
    sdst = jnp.take(dst, order)
    ssrc = jnp.take(src, order)
    bounds = jnp.arange(0, _NP + _C, _C, dtype=jnp.int32)
    cptr = jnp.searchsorted(sdst, bounds, side="left").astype(jnp.int32)
    cptr = jnp.pad(cptr, (0, _CPTR_PAD - cptr.shape[0]))
    ssrc = jnp.pad(ssrc, (0, _SG))
    order = jnp.pad(order, (0, _SG))
    sdst = jnp.pad(sdst, (0, _SG))
    xp = jnp.pad(x, ((0, _NP - _N), (0, 0)))
    s = scalars

    def layer(p, xin):
        Do = p["W_self"].shape[1]
        xm = _mm_bias(xin, p["W_msg"], p["b_msg"])
        xs = _mm_bias(xin, p["W_self"], p["b_self"])
        ea = _mm_bias(edge_attr, p["W_edge"], None)
        return _sc_edge_layer(xm, ea, xs, ssrc, order, sdst, cptr, Do)

    x1 = layer(params[0], xp)
    x2 = layer(params[1], x1)
    x2w = _wsum([s[0], s[1]], [x1, x2])
    x3 = layer(params[2], x2w)
    x3w = _wsum([s[2], s[3], s[4]], [x1, x2w, x3])
    x4 = layer(params[3], x3w)
    x4w = _wsum([s[5], s[6], s[7], s[8]], [x1, x2w, x3w, x4])
    x5 = layer(params[3], x4w)
    x5w = _wsum([s[9], s[10], s[11], s[12], s[13]], [x1, x2w, x3w, x4w, x5])
    out = layer(params[5], x5w)
    return out[:_N]


# speedup vs baseline: 2.6871x; 1.8489x over previous
import functools

import jax
import jax.numpy as jnp
from jax import lax
from jax.experimental import pallas as pl
from jax.experimental.pallas import tpu as pltpu
from jax.experimental.pallas import tpu_sc as plsc

_N = 10000
_E = 160000
_NP = 10240
_C = 64
_NCHUNK = _NP // _C
_NW = 32
_CPW = _NCHUNK // _NW
_G = 32
_CPTR_PAD = 192


def _mm_bias(a, w, b):
    M, K = a.shape
    Do = w.shape[1]
    BM = 256
    if b is None:
        b = jnp.zeros((1, Do), jnp.float32)
    else:
        b = b.reshape(1, Do)

    def kern(a_ref, w_ref, b_ref, o_ref):
        o_ref[...] = jnp.dot(a_ref[...], w_ref[...],
                             preferred_element_type=jnp.float32) + b_ref[...]

    return pl.pallas_call(
        kern,
        grid=(M // BM,),
        in_specs=[
            pl.BlockSpec((BM, K), lambda i: (i, 0)),
            pl.BlockSpec((K, Do), lambda i: (0, 0)),
            pl.BlockSpec((1, Do), lambda i: (0, 0)),
        ],
        out_specs=pl.BlockSpec((BM, Do), lambda i: (i, 0)),
        out_shape=jax.ShapeDtypeStruct((M, Do), jnp.float32),
    )(a, w, b)


def _wsum(coefs, arrays):
    M, Do = arrays[0].shape
    BM = 512
    k = len(arrays)
    cof = jnp.zeros((8,), jnp.float32).at[:k].set(jnp.stack(coefs))

    def kern(c_ref, *refs):
        o_ref = refs[-1]
        acc = c_ref[0] * refs[0][...]
        for i in range(1, k):
            acc = acc + c_ref[i] * refs[i][...]
        o_ref[...] = acc

    return pl.pallas_call(
        kern,
        grid=(M // BM,),
        in_specs=[pl.BlockSpec(memory_space=pltpu.SMEM)] +
                 [pl.BlockSpec((BM, Do), lambda i: (i, 0)) for _ in range(k)],
        out_specs=pl.BlockSpec((BM, Do), lambda i: (i, 0)),
        out_shape=jax.ShapeDtypeStruct((M, Do), jnp.float32),
    )(cof, *arrays)


def _sget(vref, i):
    v = vref[pl.ds(i, 16)]
    return v[0]


_NGS = 32
_SG = _NGS * _G
_SGP = _SG + 32


def _sc_edge_layer(xm, ea, xs, ssrc, sord, sdst, cptr, Do):
    mesh = plsc.VectorSubcoreMesh(core_axis_name="c", subcore_axis_name="s")

    @functools.partial(
        pl.kernel,
        out_type=jax.ShapeDtypeStruct((_NP, Do), jnp.float32),
        mesh=mesh,
        scratch_types=[
            pltpu.VMEM((_CPTR_PAD,), jnp.int32),
            pltpu.VMEM((_SGP,), jnp.int32),
            pltpu.VMEM((_SGP,), jnp.int32),
            pltpu.VMEM((_SGP,), jnp.int32),
            pltpu.VMEM((2, _G, Do), jnp.float32),
            pltpu.VMEM((2, _G, Do), jnp.float32),
            pltpu.VMEM((_C, Do), jnp.float32),
            pltpu.SemaphoreType.DMA,
            pltpu.SemaphoreType.DMA,
            pltpu.SemaphoreType.DMA,
            pltpu.SemaphoreType.DMA,
        ],
    )
    def k(xm_h, ea_h, xs_h, ssrc_h, sord_h, sdst_h, cptr_h, h_out,
          cptr_v, srcb, ordb, dstb, xg, eg, acc,
          semx0, seme0, semx1, seme1):
        wid = lax.axis_index("c") * 16 + lax.axis_index("s")
        pltpu.sync_copy(cptr_h, cptr_v)
        sems = ((semx0, seme0), (semx1, seme1))

        def issue(kk, slot):
            koff = kk * _G
            pltpu.async_copy(xm_h.at[srcb.at[pl.ds(koff, _G)]], xg.at[slot],
                             sems[slot][0])
            pltpu.async_copy(ea_h.at[ordb.at[pl.ds(koff, _G)]], eg.at[slot],
                             sems[slot][1])

        def wait(kk, slot):
            koff = kk * _G
            pltpu.make_async_copy(xm_h.at[srcb.at[pl.ds(koff, _G)]],
                                  xg.at[slot], sems[slot][0]).wait()
            pltpu.make_async_copy(ea_h.at[ordb.at[pl.ds(koff, _G)]],
                                  eg.at[slot], sems[slot][1]).wait()

        def chunk_body(t, _):
            chunk = wid * _CPW + t
            base_row = chunk * _C
            e_lo = _sget(cptr_v, chunk)
            e_hi = _sget(cptr_v, chunk + 1)

            @plsc.parallel_loop(0, _C, step=1, unroll=2)
            def _(r):
                for j in range(Do // 16):
                    acc[r, pl.ds(j * 16, 16)] = jnp.zeros((16,), jnp.float32)

            g_lo = e_lo // _G
            g_hi = (e_hi + _G - 1) // _G
            nsg = (g_hi - g_lo + _NGS - 1) // _NGS

            def sgbody(i, _):
                gstart = g_lo + i * _NGS
                gcnt = jnp.minimum(_NGS, g_hi - gstart)
                off = gstart * _G
                pltpu.sync_copy(ssrc_h.at[pl.ds(off, _SG)],
                                srcb.at[pl.ds(0, _SG)])
                pltpu.sync_copy(sord_h.at[pl.ds(off, _SG)],
                                ordb.at[pl.ds(0, _SG)])
                pltpu.sync_copy(sdst_h.at[pl.ds(off, _SG)],
                                dstb.at[pl.ds(0, _SG)])

                @pl.when(gcnt > 0)
                def _():
                    issue(0, 0)

                def compute(kk, slot):
                    koff = kk * _G
                    goff = off + koff
                    lo = jnp.maximum(goff, e_lo) - goff
                    hi = jnp.minimum(goff + _G, e_hi) - goff

                    @plsc.parallel_loop(lo, hi, step=1, unroll=2)
                    def _(t2):
                        row = _sget(dstb, koff + t2) - base_row
                        for j in range(Do // 16):
                            sl = pl.ds(j * 16, 16)
                            v = jnp.maximum(xg[slot, t2, sl] + eg[slot, t2, sl],
                                            0.0)
                            plsc.addupdate(acc.at[row, sl], v)

                def step(kk, slot):
                    wait(kk, slot)

                    @pl.when(kk + 1 < gcnt)
                    def _():
                        issue(kk + 1, 1 - slot)
                    compute(kk, slot)

                def kbody(kk, _):
                    @pl.when(kk % 2 == 0)
                    def _():
                        step(kk, 0)

                    @pl.when(kk % 2 == 1)
                    def _():
                        step(kk, 1)
                    return 0
                lax.fori_loop(0, gcnt, kbody, 0)
                return 0
            lax.fori_loop(0, nsg, sgbody, 0)

            pltpu.sync_copy(xs_h.at[pl.ds(base_row, _C // 2)], xg.at[0])
            pltpu.sync_copy(xs_h.at[pl.ds(base_row + _C // 2, _C // 2)],
                            xg.at[1])

            @plsc.parallel_loop(0, _C, step=1, unroll=2)
            def _(r):
                b = r // (_C // 2)
                rr = r - b * (_C // 2)
                for j in range(Do // 16):
                    sl = pl.ds(j * 16, 16)
                    acc[r, sl] = jnp.maximum(acc[r, sl] + xg[b, rr, sl], 0.0)
            pltpu.sync_copy(acc, h_out.at[pl.ds(base_row, _C)])
            return 0
        lax.fori_loop(0, _CPW, chunk_body, 0)

    return k(xm, ea, xs, ssrc, sord, sdst, cptr)


def kernel(x, edge_index, edge_attr, params, scalars):
    src = edge_index[0].astype(jnp.int32)
    dst = edge_index[1].astype(jnp.int32)
    order = jnp.argsort(dst).astype(jnp.int32)
    sdst = jnp.take(dst, order)
    ssrc = jnp.take(src, order)
    bounds = jnp.arange(0, _NP + _C, _C, dtype=jnp.int32)
    cptr = jnp.searchsorted(sdst, bounds, side="left").astype(jnp.int32)
    cptr = jnp.pad(cptr, (0, _CPTR_PAD - cptr.shape[0]))
    ssrc = jnp.pad(ssrc, (0, _SG))
    order = jnp.pad(order, (0, _SG))
    sdst = jnp.pad(sdst, (0, _SG))
    xp = jnp.pad(x, ((0, _NP - _N), (0, 0)))
    s = scalars

    def layer(p, xin):
        Do = p["W_self"].shape[1]
        xm = _mm_bias(xin, p["W_msg"], p["b_msg"])
        xs = _mm_bias(xin, p["W_self"], p["b_self"])
        ea = _mm_bias(edge_attr, p["W_edge"], None)
        return _sc_edge_layer(xm, ea, xs, ssrc, order, sdst, cptr, Do)

    x1 = layer(params[0], xp)
    x2 = layer(params[1], x1)
    x2w = _wsum([s[0], s[1]], [x1, x2])
    x3 = layer(params[2], x2w)
    x3w = _wsum([s[2], s[3], s[4]], [x1, x2w, x3])
    x4 = layer(params[3], x3w)
    x4w = _wsum([s[5], s[6], s[7], s[8]], [x1, x2w, x3w, x4])
    x5 = layer(params[3], x4w)
    x5w = _wsum([s[9], s[10], s[11], s[12], s[13]], [x1, x2w, x3w, x4w, x5])
    out = layer(params[5], x5w)
    return out[:_N]
